# baseline (speedup 1.0000x reference)
import functools

import jax
import jax.numpy as jnp
from jax import lax
from jax.experimental import pallas as pl
from jax.experimental.pallas import tpu as pltpu
from jax.experimental.pallas import tpu_sc as plsc

N_NODES = 10000
N_EDGES = 320000
ATOM_FDIM = 128
BOND_FDIM = 16
HIDDEN = 300
HPAD = 320
NQ = 4
QW = HPAD // NQ
DEPTH = 3
BR = 1000
NB = N_NODES // BR

NC, NS = 2, 16
G = 128
EPAD = 327680
NGRP = EPAD // G
GPT = NGRP // NS
SCH = 16
NSC = GPT // SCH
NBUF = 2
NPAD = 10112
DUMMY = 10000
RPT = NPAD // NS
GPT_F = NGRP // (NC * NS)
NSC_F = GPT_F // SCH


def _mm_init_body(x_ref, wi_ref, inp_ref, mq_ref):
    v = jnp.dot(x_ref[...], wi_ref[0], preferred_element_type=jnp.float32)
    inp_ref[0] = v
    mq_ref[0] = jnp.maximum(v, 0.0)


def _mm_round_body(inp_ref, nfa_ref, nfb_ref, n4_ref, wh2_ref, wh1_ref, mq_ref):
    nf = nfa_ref[...] + nfb_ref[...]
    v = (inp_ref[0]
         + jnp.dot(nf, wh2_ref[0], preferred_element_type=jnp.float32))
    w = wh1_ref[0]
    for p in range(NQ):
        v = v + jnp.dot(n4_ref[p], w[p], preferred_element_type=jnp.float32)
    mq_ref[0] = jnp.maximum(v, 0.0)


def _mm_out_body(x_ref, n4_ref, wo1_ref, wo2_ref, o_ref):
    v = jnp.dot(x_ref[...], wo1_ref[...], preferred_element_type=jnp.float32)
    for p in range(NQ):
        v = v + jnp.dot(n4_ref[p], wo2_ref[p], preferred_element_type=jnp.float32)
    o_ref[...] = jnp.maximum(v, 0.0)


def _mm_init(x, wi4):
    return pl.pallas_call(
        _mm_init_body,
        grid=(NB, NQ),
        in_specs=[pl.BlockSpec((BR, ATOM_FDIM), lambda i, q: (i, 0)),
                  pl.BlockSpec((1, ATOM_FDIM, QW), lambda i, q: (q, 0, 0))],
        out_specs=[pl.BlockSpec((1, BR, QW), lambda i, q: (q, i, 0)),
                   pl.BlockSpec((1, BR, QW), lambda i, q: (q, i, 0))],
        out_shape=[jax.ShapeDtypeStruct((NQ, N_NODES, QW), jnp.float32),
                   jax.ShapeDtypeStruct((NQ, NPAD, QW), jnp.float32)],
    )(x, wi4)


def _mm_round(inp, nfa, nfb, n4, wh2, wh1):
    return pl.pallas_call(
        _mm_round_body,
        grid=(NB, NQ),
        in_specs=[pl.BlockSpec((1, BR, QW), lambda i, q: (q, i, 0)),
                  pl.BlockSpec((BR, BOND_FDIM), lambda i, q: (i, 0)),
                  pl.BlockSpec((BR, BOND_FDIM), lambda i, q: (i, 0)),
                  pl.BlockSpec((NQ, BR, QW), lambda i, q: (0, i, 0)),
                  pl.BlockSpec((1, BOND_FDIM, QW), lambda i, q: (q, 0, 0)),
                  pl.BlockSpec((1, NQ, QW, QW), lambda i, q: (q, 0, 0, 0))],
        out_specs=pl.BlockSpec((1, BR, QW), lambda i, q: (q, i, 0)),
        out_shape=jax.ShapeDtypeStruct((NQ, NPAD, QW), jnp.float32),
    )(inp, nfa, nfb, n4, wh2, wh1)


def _mm_out(x, n4, wo1, wo2):
    return pl.pallas_call(
        _mm_out_body,
        grid=(NB,),
        in_specs=[pl.BlockSpec((BR, ATOM_FDIM), lambda i: (i, 0)),
                  pl.BlockSpec((NQ, BR, QW), lambda i: (0, i, 0)),
                  pl.BlockSpec((ATOM_FDIM, HIDDEN), lambda i: (0, 0)),
                  pl.BlockSpec((NQ, QW, HIDDEN), lambda i: (0, 0, 0))],
        out_specs=pl.BlockSpec((BR, HIDDEN), lambda i: (i, 0)),
        out_shape=jax.ShapeDtypeStruct((N_NODES, HIDDEN), jnp.float32),
    )(x, n4, wo1, wo2)


_MESH = plsc.VectorSubcoreMesh(core_axis_name="c", subcore_axis_name="s")


def _zero_fill(zbuf, rows, width):
    for i in range(rows):
        for j in range(width // 16):
            zbuf[i, pl.ds(j * 16, 16)] = jnp.zeros((16,), jnp.float32)


def _segsum_body(mq, src2, dst2, n4, tbl, acc, srcv, dstv, ring, zbuf,
                 gsem, ssem):
    c = lax.axis_index("c")
    s = lax.axis_index("s")
    zbase = s * RPT

    _zero_fill(zbuf, 8, QW)

    for q in range(NC):
        qq = c * NC + q

        def zloop(k, _):
            pltpu.sync_copy(zbuf, acc.at[pl.ds(zbase + k * 8, 8)])
            return 0
        lax.fori_loop(0, RPT // 8, zloop, 0)
        pltpu.sync_copy(mq.at[qq, pl.ds(zbase, RPT)],
                        tbl.at[pl.ds(zbase, RPT)])
        plsc.subcore_barrier()

        def schunk(sc_i, _):
            gb = s * GPT + sc_i * SCH
            pltpu.sync_copy(src2.at[pl.ds(gb, SCH)], srcv)
            pltpu.sync_copy(dst2.at[pl.ds(gb, SCH)], dstv)

            for b in range(NBUF):
                pltpu.async_copy(tbl.at[srcv.at[b]], ring.at[b], gsem.at[b])

            def mloop(k2, _):
                for b in range(NBUF):
                    j = k2 * NBUF + b
                    pltpu.make_async_copy(mq.at[0, pl.ds(0, G)], ring.at[b],
                                          gsem.at[b]).wait()
                    pltpu.async_copy(ring.at[b], acc.at[dstv.at[j]],
                                     ssem.at[b], add=True)
                    bp = (b - 1) % NBUF
                    @pl.when((j >= 1) & (j - 1 + NBUF < SCH))
                    def _():
                        pltpu.make_async_copy(mq.at[0, pl.ds(0, G)],
                                              ring.at[bp], ssem.at[bp]).wait()
                        pltpu.async_copy(tbl.at[srcv.at[j - 1 + NBUF]],
                                         ring.at[bp], gsem.at[bp])
                return 0
            lax.fori_loop(0, SCH // NBUF, mloop, 0)

            for b in range(NBUF):
                pltpu.make_async_copy(mq.at[0, pl.ds(0, G)], ring.at[b],
                                      ssem.at[b]).wait()
            return 0
        lax.fori_loop(0, NSC, schunk, 0)
        plsc.subcore_barrier()

        pltpu.sync_copy(acc.at[pl.ds(zbase, RPT)],
                        n4.at[qq, pl.ds(zbase, RPT)])
        plsc.subcore_barrier()


_segsum_sc = functools.partial(
    pl.kernel,
    out_type=jax.ShapeDtypeStruct((NQ, NPAD, QW), jnp.float32),
    mesh=_MESH,
    compiler_params=pltpu.CompilerParams(use_tc_tiling_on_sc=False),
    scratch_types=[
        pltpu.VMEM_SHARED((NPAD, QW), jnp.float32),
        pltpu.VMEM_SHARED((NPAD, QW), jnp.float32),
        pltpu.VMEM((SCH, G), jnp.int32),
        pltpu.VMEM((SCH, G), jnp.int32),
        pltpu.VMEM((NBUF, G, QW), jnp.float32),
        pltpu.VMEM((8, QW), jnp.float32),
        pltpu.SemaphoreType.DMA((NBUF,)),
        pltpu.SemaphoreType.DMA((NBUF,)),
    ],
)(_segsum_body)


def _bond_body(ea, dst2, nfa, nfb, acc, dstv, ring, zbuf, gsem, ssem):
    c = lax.axis_index("c")
    s = lax.axis_index("s")
    zbase = s * RPT

    _zero_fill(zbuf, 8, BOND_FDIM)
    def zloop(k, _):
        pltpu.sync_copy(zbuf, acc.at[pl.ds(zbase + k * 8, 8)])
        return 0
    lax.fori_loop(0, RPT // 8, zloop, 0)
    plsc.subcore_barrier()

    def schunk(sc_i, _):
        gb = (c * NS + s) * GPT_F + sc_i * SCH
        pltpu.sync_copy(dst2.at[pl.ds(gb, SCH)], dstv)

        for b in range(NBUF):
            pltpu.async_copy(ea.at[pl.ds((gb + b) * G, G)], ring.at[b],
                             gsem.at[b])

        def mloop(k2, _):
            for b in range(NBUF):
                j = k2 * NBUF + b
                pltpu.make_async_copy(ea.at[pl.ds(0, G)], ring.at[b],
                                      gsem.at[b]).wait()
                pltpu.async_copy(ring.at[b], acc.at[dstv.at[j]], ssem.at[b],
                                 add=True)
                bp = (b - 1) % NBUF
                @pl.when((j >= 1) & (j - 1 + NBUF < SCH))
                def _():
                    pltpu.make_async_copy(ea.at[pl.ds(0, G)], ring.at[bp],
                                          ssem.at[bp]).wait()
                    pltpu.async_copy(ea.at[pl.ds((gb + j - 1 + NBUF) * G, G)],
                                     ring.at[bp], gsem.at[bp])
            return 0
        lax.fori_loop(0, SCH // NBUF, mloop, 0)

        for b in range(NBUF):
            pltpu.make_async_copy(ea.at[pl.ds(0, G)], ring.at[b],
                                  ssem.at[b]).wait()
        return 0
    lax.fori_loop(0, NSC_F, schunk, 0)
    plsc.subcore_barrier()

    @pl.when(c == 0)
    def _():
        pltpu.sync_copy(acc.at[pl.ds(zbase, RPT)], nfa.at[pl.ds(zbase, RPT)])
    @pl.when(c == 1)
    def _():
        pltpu.sync_copy(acc.at[pl.ds(zbase, RPT)], nfb.at[pl.ds(zbase, RPT)])


_bond_sc = functools.partial(
    pl.kernel,
    out_type=[jax.ShapeDtypeStruct((NPAD, BOND_FDIM), jnp.float32),
              jax.ShapeDtypeStruct((NPAD, BOND_FDIM), jnp.float32)],
    mesh=_MESH,
    compiler_params=pltpu.CompilerParams(use_tc_tiling_on_sc=False),
    scratch_types=[
        pltpu.VMEM_SHARED((NPAD, BOND_FDIM), jnp.float32),
        pltpu.VMEM((SCH, G), jnp.int32),
        pltpu.VMEM((NBUF, G, BOND_FDIM), jnp.float32),
        pltpu.VMEM((8, BOND_FDIM), jnp.float32),
        pltpu.SemaphoreType.DMA((NBUF,)),
        pltpu.SemaphoreType.DMA((NBUF,)),
    ],
)(_bond_body)


def kernel(x, edge_index, edge_attr, W_i, W_h, W_o):
    src = edge_index[0].astype(jnp.int32)
    dst = edge_index[1].astype(jnp.int32)

    src2 = jnp.pad(src, (0, EPAD - N_EDGES)).reshape(NGRP, G)
    dst2 = jnp.pad(dst, (0, EPAD - N_EDGES),
                   constant_values=DUMMY).reshape(NGRP, G)
    ea_pad = jnp.pad(edge_attr, ((0, EPAD - N_EDGES), (0, 0)))

    wi4 = jnp.pad(W_i, ((0, 0), (0, HPAD - HIDDEN)))
    wi4 = wi4.reshape(ATOM_FDIM, NQ, QW).transpose(1, 0, 2)
    wh1 = jnp.pad(W_h[:HIDDEN],
                  ((0, HPAD - HIDDEN), (0, HPAD - HIDDEN)))
    wh1 = wh1.reshape(NQ, QW, NQ, QW).transpose(2, 0, 1, 3)
    wh2 = jnp.pad(W_h[HIDDEN:], ((0, 0), (0, HPAD - HIDDEN)))
    wh2 = wh2.reshape(BOND_FDIM, NQ, QW).transpose(1, 0, 2)
    wo1 = W_o[:ATOM_FDIM]
    wo2 = jnp.pad(W_o[ATOM_FDIM:], ((0, HPAD - HIDDEN), (0, 0)))
    wo2 = wo2.reshape(NQ, QW, HIDDEN)

    inp, mq = _mm_init(x, wi4)
    nfa, nfb = _bond_sc(ea_pad, dst2)

    for _ in range(DEPTH - 1):
        n4 = _segsum_sc(mq, src2, dst2)
        mq = _mm_round(inp, nfa, nfb, n4, wh2, wh1)

    n4 = _segsum_sc(mq, src2, dst2)
    return _mm_out(x, n4, wo1, wo2)


# speedup vs baseline: 1.1001x; 1.1001x over previous
import functools

import jax
import jax.numpy as jnp
from jax import lax
from jax.experimental import pallas as pl
from jax.experimental.pallas import tpu as pltpu
from jax.experimental.pallas import tpu_sc as plsc

N_NODES = 10000
N_EDGES = 320000
ATOM_FDIM = 128
BOND_FDIM = 16
HIDDEN = 300
HPAD = 320
NQ = 4
QW = HPAD // NQ
DEPTH = 3
BR = 1000
NB = N_NODES // BR

NC, NS = 2, 16
G = 32
EPAD = 327680
NGRP = EPAD // G
GPT = NGRP // NS
SCH = 32
NSC = GPT // SCH
NBUF = 8
NPAD = 10112
DUMMY = 10000
RPT = NPAD // NS
GPT_F = NGRP // (NC * NS)
NSC_F = GPT_F // SCH


def _mm_init_body(x_ref, wi_ref, inp_ref, mq_ref):
    v = jnp.dot(x_ref[...], wi_ref[0], preferred_element_type=jnp.float32)
    inp_ref[0] = v
    mq_ref[0] = jnp.maximum(v, 0.0)


def _mm_round_body(inp_ref, nfa_ref, nfb_ref, n4_ref, wh2_ref, wh1_ref, mq_ref):
    nf = nfa_ref[...] + nfb_ref[...]
    v = (inp_ref[0]
         + jnp.dot(nf, wh2_ref[0], preferred_element_type=jnp.float32))
    w = wh1_ref[0]
    for p in range(NQ):
        v = v + jnp.dot(n4_ref[p], w[p], preferred_element_type=jnp.float32)
    mq_ref[0] = jnp.maximum(v, 0.0)


def _mm_out_body(x_ref, n4_ref, wo1_ref, wo2_ref, o_ref):
    v = jnp.dot(x_ref[...], wo1_ref[...], preferred_element_type=jnp.float32)
    for p in range(NQ):
        v = v + jnp.dot(n4_ref[p], wo2_ref[p], preferred_element_type=jnp.float32)
    o_ref[...] = jnp.maximum(v, 0.0)


def _mm_init(x, wi4):
    return pl.pallas_call(
        _mm_init_body,
        grid=(NB, NQ),
        in_specs=[pl.BlockSpec((BR, ATOM_FDIM), lambda i, q: (i, 0)),
                  pl.BlockSpec((1, ATOM_FDIM, QW), lambda i, q: (q, 0, 0))],
        out_specs=[pl.BlockSpec((1, BR, QW), lambda i, q: (q, i, 0)),
                   pl.BlockSpec((1, BR, QW), lambda i, q: (q, i, 0))],
        out_shape=[jax.ShapeDtypeStruct((NQ, N_NODES, QW), jnp.float32),
                   jax.ShapeDtypeStruct((NQ, NPAD, QW), jnp.float32)],
    )(x, wi4)


def _mm_round(inp, nfa, nfb, n4, wh2, wh1):
    return pl.pallas_call(
        _mm_round_body,
        grid=(NB, NQ),
        in_specs=[pl.BlockSpec((1, BR, QW), lambda i, q: (q, i, 0)),
                  pl.BlockSpec((BR, BOND_FDIM), lambda i, q: (i, 0)),
                  pl.BlockSpec((BR, BOND_FDIM), lambda i, q: (i, 0)),
                  pl.BlockSpec((NQ, BR, QW), lambda i, q: (0, i, 0)),
                  pl.BlockSpec((1, BOND_FDIM, QW), lambda i, q: (q, 0, 0)),
                  pl.BlockSpec((1, NQ, QW, QW), lambda i, q: (q, 0, 0, 0))],
        out_specs=pl.BlockSpec((1, BR, QW), lambda i, q: (q, i, 0)),
        out_shape=jax.ShapeDtypeStruct((NQ, NPAD, QW), jnp.float32),
    )(inp, nfa, nfb, n4, wh2, wh1)


def _mm_out(x, n4, wo1, wo2):
    return pl.pallas_call(
        _mm_out_body,
        grid=(NB,),
        in_specs=[pl.BlockSpec((BR, ATOM_FDIM), lambda i: (i, 0)),
                  pl.BlockSpec((NQ, BR, QW), lambda i: (0, i, 0)),
                  pl.BlockSpec((ATOM_FDIM, HIDDEN), lambda i: (0, 0)),
                  pl.BlockSpec((NQ, QW, HIDDEN), lambda i: (0, 0, 0))],
        out_specs=pl.BlockSpec((BR, HIDDEN), lambda i: (i, 0)),
        out_shape=jax.ShapeDtypeStruct((N_NODES, HIDDEN), jnp.float32),
    )(x, n4, wo1, wo2)


_MESH = plsc.VectorSubcoreMesh(core_axis_name="c", subcore_axis_name="s")


def _zero_fill(zbuf, rows, width):
    for i in range(rows):
        for j in range(width // 16):
            zbuf[i, pl.ds(j * 16, 16)] = jnp.zeros((16,), jnp.float32)


def _segsum_body(mq, src2, dst2, n4, tbl, acc, srcv, dstv, ring, zbuf,
                 gsem, ssem):
    c = lax.axis_index("c")
    s = lax.axis_index("s")
    zbase = s * RPT

    _zero_fill(zbuf, 8, QW)

    for q in range(NC):
        qq = c * NC + q

        def zloop(k, _):
            pltpu.sync_copy(zbuf, acc.at[pl.ds(zbase + k * 8, 8)])
            return 0
        lax.fori_loop(0, RPT // 8, zloop, 0)
        pltpu.sync_copy(mq.at[qq, pl.ds(zbase, RPT)],
                        tbl.at[pl.ds(zbase, RPT)])
        plsc.subcore_barrier()

        def schunk(sc_i, _):
            gb = s * GPT + sc_i * SCH
            pltpu.sync_copy(src2.at[pl.ds(gb, SCH)], srcv)
            pltpu.sync_copy(dst2.at[pl.ds(gb, SCH)], dstv)

            for b in range(NBUF):
                pltpu.async_copy(tbl.at[srcv.at[b]], ring.at[b], gsem.at[b])

            def mloop(k2, _):
                for b in range(NBUF):
                    j = k2 * NBUF + b
                    pltpu.make_async_copy(mq.at[0, pl.ds(0, G)], ring.at[b],
                                          gsem.at[b]).wait()
                    pltpu.async_copy(ring.at[b], acc.at[dstv.at[j]],
                                     ssem.at[b], add=True)
                    bp = (b - 1) % NBUF
                    @pl.when((j >= 1) & (j - 1 + NBUF < SCH))
                    def _():
                        pltpu.make_async_copy(mq.at[0, pl.ds(0, G)],
                                              ring.at[bp], ssem.at[bp]).wait()
                        pltpu.async_copy(tbl.at[srcv.at[j - 1 + NBUF]],
                                         ring.at[bp], gsem.at[bp])
                return 0
            lax.fori_loop(0, SCH // NBUF, mloop, 0)

            for b in range(NBUF):
                pltpu.make_async_copy(mq.at[0, pl.ds(0, G)], ring.at[b],
                                      ssem.at[b]).wait()
            return 0
        lax.fori_loop(0, NSC, schunk, 0)
        plsc.subcore_barrier()

        pltpu.sync_copy(acc.at[pl.ds(zbase, RPT)],
                        n4.at[qq, pl.ds(zbase, RPT)])
        plsc.subcore_barrier()


_segsum_sc = functools.partial(
    pl.kernel,
    out_type=jax.ShapeDtypeStruct((NQ, NPAD, QW), jnp.float32),
    mesh=_MESH,
    compiler_params=pltpu.CompilerParams(use_tc_tiling_on_sc=False),
    scratch_types=[
        pltpu.VMEM_SHARED((NPAD, QW), jnp.float32),
        pltpu.VMEM_SHARED((NPAD, QW), jnp.float32),
        pltpu.VMEM((SCH, G), jnp.int32),
        pltpu.VMEM((SCH, G), jnp.int32),
        pltpu.VMEM((NBUF, G, QW), jnp.float32),
        pltpu.VMEM((8, QW), jnp.float32),
        pltpu.SemaphoreType.DMA((NBUF,)),
        pltpu.SemaphoreType.DMA((NBUF,)),
    ],
)(_segsum_body)


def _bond_body(ea, dst2, nfa, nfb, acc, dstv, ring, zbuf, gsem, ssem):
    c = lax.axis_index("c")
    s = lax.axis_index("s")
    zbase = s * RPT

    _zero_fill(zbuf, 8, BOND_FDIM)
    def zloop(k, _):
        pltpu.sync_copy(zbuf, acc.at[pl.ds(zbase + k * 8, 8)])
        return 0
    lax.fori_loop(0, RPT // 8, zloop, 0)
    plsc.subcore_barrier()

    def schunk(sc_i, _):
        gb = (c * NS + s) * GPT_F + sc_i * SCH
        pltpu.sync_copy(dst2.at[pl.ds(gb, SCH)], dstv)

        for b in range(NBUF):
            pltpu.async_copy(ea.at[pl.ds((gb + b) * G, G)], ring.at[b],
                             gsem.at[b])

        def mloop(k2, _):
            for b in range(NBUF):
                j = k2 * NBUF + b
                pltpu.make_async_copy(ea.at[pl.ds(0, G)], ring.at[b],
                                      gsem.at[b]).wait()
                pltpu.async_copy(ring.at[b], acc.at[dstv.at[j]], ssem.at[b],
                                 add=True)
                bp = (b - 1) % NBUF
                @pl.when((j >= 1) & (j - 1 + NBUF < SCH))
                def _():
                    pltpu.make_async_copy(ea.at[pl.ds(0, G)], ring.at[bp],
                                          ssem.at[bp]).wait()
                    pltpu.async_copy(ea.at[pl.ds((gb + j - 1 + NBUF) * G, G)],
                                     ring.at[bp], gsem.at[bp])
            return 0
        lax.fori_loop(0, SCH // NBUF, mloop, 0)

        for b in range(NBUF):
            pltpu.make_async_copy(ea.at[pl.ds(0, G)], ring.at[b],
                                  ssem.at[b]).wait()
        return 0
    lax.fori_loop(0, NSC_F, schunk, 0)
    plsc.subcore_barrier()

    @pl.when(c == 0)
    def _():
        pltpu.sync_copy(acc.at[pl.ds(zbase, RPT)], nfa.at[pl.ds(zbase, RPT)])
    @pl.when(c == 1)
    def _():
        pltpu.sync_copy(acc.at[pl.ds(zbase, RPT)], nfb.at[pl.ds(zbase, RPT)])


_bond_sc = functools.partial(
    pl.kernel,
    out_type=[jax.ShapeDtypeStruct((NPAD, BOND_FDIM), jnp.float32),
              jax.ShapeDtypeStruct((NPAD, BOND_FDIM), jnp.float32)],
    mesh=_MESH,
    compiler_params=pltpu.CompilerParams(use_tc_tiling_on_sc=False),
    scratch_types=[
        pltpu.VMEM_SHARED((NPAD, BOND_FDIM), jnp.float32),
        pltpu.VMEM((SCH, G), jnp.int32),
        pltpu.VMEM((NBUF, G, BOND_FDIM), jnp.float32),
        pltpu.VMEM((8, BOND_FDIM), jnp.float32),
        pltpu.SemaphoreType.DMA((NBUF,)),
        pltpu.SemaphoreType.DMA((NBUF,)),
    ],
)(_bond_body)


def kernel(x, edge_index, edge_attr, W_i, W_h, W_o):
    src = edge_index[0].astype(jnp.int32)
    dst = edge_index[1].astype(jnp.int32)

    src2 = jnp.pad(src, (0, EPAD - N_EDGES)).reshape(NGRP, G)
    dst2 = jnp.pad(dst, (0, EPAD - N_EDGES),
                   constant_values=DUMMY).reshape(NGRP, G)
    ea_pad = jnp.pad(edge_attr, ((0, EPAD - N_EDGES), (0, 0)))

    wi4 = jnp.pad(W_i, ((0, 0), (0, HPAD - HIDDEN)))
    wi4 = wi4.reshape(ATOM_FDIM, NQ, QW).transpose(1, 0, 2)
    wh1 = jnp.pad(W_h[:HIDDEN],
                  ((0, HPAD - HIDDEN), (0, HPAD - HIDDEN)))
    wh1 = wh1.reshape(NQ, QW, NQ, QW).transpose(2, 0, 1, 3)
    wh2 = jnp.pad(W_h[HIDDEN:], ((0, 0), (0, HPAD - HIDDEN)))
    wh2 = wh2.reshape(BOND_FDIM, NQ, QW).transpose(1, 0, 2)
    wo1 = W_o[:ATOM_FDIM]
    wo2 = jnp.pad(W_o[ATOM_FDIM:], ((0, HPAD - HIDDEN), (0, 0)))
    wo2 = wo2.reshape(NQ, QW, HIDDEN)

    inp, mq = _mm_init(x, wi4)
    nfa, nfb = _bond_sc(ea_pad, dst2)

    for _ in range(DEPTH - 1):
        n4 = _segsum_sc(mq, src2, dst2)
        mq = _mm_round(inp, nfa, nfb, n4, wh2, wh1)

    n4 = _segsum_sc(mq, src2, dst2)
    return _mm_out(x, n4, wo1, wo2)


# speedup vs baseline: 1.2017x; 1.0924x over previous
import functools

import jax
import jax.numpy as jnp
from jax import lax
from jax.experimental import pallas as pl
from jax.experimental.pallas import tpu as pltpu
from jax.experimental.pallas import tpu_sc as plsc

N_NODES = 10000
N_EDGES = 320000
ATOM_FDIM = 128
BOND_FDIM = 16
HIDDEN = 300
HPAD = 320
NQ = 4
QW = HPAD // NQ
DEPTH = 3
BR = 1000
NB = N_NODES // BR

NC, NS = 2, 16
G = 64
EPAD = 327680
NGRP = EPAD // G
GPT = NGRP // NS
SCH = 32
NSC = GPT // SCH
NBUF = 4
NPAD = 10112
DUMMY = 10000
RPT = NPAD // NS
GPT_F = NGRP // (NC * NS)
NSC_F = GPT_F // SCH
NGRP_REAL = N_EDGES // G


def _mm_init_body(x_ref, wi_ref, inp_ref, mq_ref):
    v = jnp.dot(x_ref[...], wi_ref[0], preferred_element_type=jnp.float32)
    inp_ref[0] = v
    mq_ref[0] = jnp.maximum(v, 0.0)


def _mm_round_body(inp_ref, nfa_ref, nfb_ref, n4_ref, wh2_ref, wh1_ref, mq_ref):
    nf = nfa_ref[...] + nfb_ref[...]
    v = (inp_ref[0]
         + jnp.dot(nf, wh2_ref[0], preferred_element_type=jnp.float32))
    w = wh1_ref[0]
    for p in range(NQ):
        v = v + jnp.dot(n4_ref[p], w[p], preferred_element_type=jnp.float32)
    mq_ref[0] = jnp.maximum(v, 0.0)


def _mm_out_body(x_ref, n4_ref, wo1_ref, wo2_ref, o_ref):
    v = jnp.dot(x_ref[...], wo1_ref[...], preferred_element_type=jnp.float32)
    for p in range(NQ):
        v = v + jnp.dot(n4_ref[p], wo2_ref[p], preferred_element_type=jnp.float32)
    o_ref[...] = jnp.maximum(v, 0.0)


def _mm_init(x, wi4):
    return pl.pallas_call(
        _mm_init_body,
        grid=(NB, NQ),
        in_specs=[pl.BlockSpec((BR, ATOM_FDIM), lambda i, q: (i, 0)),
                  pl.BlockSpec((1, ATOM_FDIM, QW), lambda i, q: (q, 0, 0))],
        out_specs=[pl.BlockSpec((1, BR, QW), lambda i, q: (q, i, 0)),
                   pl.BlockSpec((1, BR, QW), lambda i, q: (q, i, 0))],
        out_shape=[jax.ShapeDtypeStruct((NQ, N_NODES, QW), jnp.float32),
                   jax.ShapeDtypeStruct((NQ, NPAD, QW), jnp.float32)],
    )(x, wi4)


def _mm_round(inp, nfa, nfb, n4, wh2, wh1):
    return pl.pallas_call(
        _mm_round_body,
        grid=(NB, NQ),
        in_specs=[pl.BlockSpec((1, BR, QW), lambda i, q: (q, i, 0)),
                  pl.BlockSpec((BR, BOND_FDIM), lambda i, q: (i, 0)),
                  pl.BlockSpec((BR, BOND_FDIM), lambda i, q: (i, 0)),
                  pl.BlockSpec((NQ, BR, QW), lambda i, q: (0, i, 0)),
                  pl.BlockSpec((1, BOND_FDIM, QW), lambda i, q: (q, 0, 0)),
                  pl.BlockSpec((1, NQ, QW, QW), lambda i, q: (q, 0, 0, 0))],
        out_specs=pl.BlockSpec((1, BR, QW), lambda i, q: (q, i, 0)),
        out_shape=jax.ShapeDtypeStruct((NQ, NPAD, QW), jnp.float32),
    )(inp, nfa, nfb, n4, wh2, wh1)


def _mm_out(x, n4, wo1, wo2):
    return pl.pallas_call(
        _mm_out_body,
        grid=(NB,),
        in_specs=[pl.BlockSpec((BR, ATOM_FDIM), lambda i: (i, 0)),
                  pl.BlockSpec((NQ, BR, QW), lambda i: (0, i, 0)),
                  pl.BlockSpec((ATOM_FDIM, HIDDEN), lambda i: (0, 0)),
                  pl.BlockSpec((NQ, QW, HIDDEN), lambda i: (0, 0, 0))],
        out_specs=pl.BlockSpec((BR, HIDDEN), lambda i: (i, 0)),
        out_shape=jax.ShapeDtypeStruct((N_NODES, HIDDEN), jnp.float32),
    )(x, n4, wo1, wo2)


_MESH = plsc.VectorSubcoreMesh(core_axis_name="c", subcore_axis_name="s")


def _segsum_body(mq, src2, dst2, zq, n4, tbl, acc, srcv, dstv, ring,
                 gsem, ssem):
    c = lax.axis_index("c")
    s = lax.axis_index("s")
    zbase = s * RPT

    for q in range(NC):
        qq = c * NC + q

        pltpu.sync_copy(zq, acc.at[pl.ds(zbase, RPT)])
        pltpu.sync_copy(mq.at[qq, pl.ds(zbase, RPT)],
                        tbl.at[pl.ds(zbase, RPT)])
        plsc.subcore_barrier()

        def schunk(sc_i, _):
            gb = s * GPT + sc_i * SCH
            pltpu.sync_copy(src2.at[pl.ds(gb, SCH)], srcv)
            pltpu.sync_copy(dst2.at[pl.ds(gb, SCH)], dstv)

            for b in range(NBUF):
                pltpu.async_copy(tbl.at[srcv.at[b]], ring.at[b], gsem.at[b])

            def mloop(k2, _):
                for b in range(NBUF):
                    j = k2 * NBUF + b
                    pltpu.make_async_copy(mq.at[0, pl.ds(0, G)], ring.at[b],
                                          gsem.at[b]).wait()
                    pltpu.async_copy(ring.at[b], acc.at[dstv.at[j]],
                                     ssem.at[b], add=True)
                    bp = (b - 1) % NBUF
                    @pl.when((j >= 1) & (j - 1 + NBUF < SCH))
                    def _():
                        pltpu.make_async_copy(mq.at[0, pl.ds(0, G)],
                                              ring.at[bp], ssem.at[bp]).wait()
                        pltpu.async_copy(tbl.at[srcv.at[j - 1 + NBUF]],
                                         ring.at[bp], gsem.at[bp])
                return 0
            lax.fori_loop(0, SCH // NBUF, mloop, 0)

            for b in range(NBUF):
                pltpu.make_async_copy(mq.at[0, pl.ds(0, G)], ring.at[b],
                                      ssem.at[b]).wait()
            return 0
        lax.fori_loop(0, NSC, schunk, 0)
        plsc.subcore_barrier()

        pltpu.sync_copy(acc.at[pl.ds(zbase, RPT)],
                        n4.at[qq, pl.ds(zbase, RPT)])
        plsc.subcore_barrier()


_segsum_sc = functools.partial(
    pl.kernel,
    out_type=jax.ShapeDtypeStruct((NQ, NPAD, QW), jnp.float32),
    mesh=_MESH,
    compiler_params=pltpu.CompilerParams(use_tc_tiling_on_sc=False),
    scratch_types=[
        pltpu.VMEM_SHARED((NPAD, QW), jnp.float32),
        pltpu.VMEM_SHARED((NPAD, QW), jnp.float32),
        pltpu.VMEM((SCH, G), jnp.int32),
        pltpu.VMEM((SCH, G), jnp.int32),
        pltpu.VMEM((NBUF, G, QW), jnp.float32),
        pltpu.SemaphoreType.DMA((NBUF,)),
        pltpu.SemaphoreType.DMA((NBUF,)),
    ],
)(_segsum_body)


def _bond_body(ea, dst2, zf, nfa, nfb, acc, dstv, ring, gsem, ssem):
    c = lax.axis_index("c")
    s = lax.axis_index("s")
    zbase = s * RPT

    pltpu.sync_copy(zf, acc.at[pl.ds(zbase, RPT)])
    plsc.subcore_barrier()

    def _ea_off(g):
        return jnp.minimum(g, NGRP_REAL - 1) * G

    def schunk(sc_i, _):
        gb = (c * NS + s) * GPT_F + sc_i * SCH
        pltpu.sync_copy(dst2.at[pl.ds(gb, SCH)], dstv)

        for b in range(NBUF):
            pltpu.async_copy(ea.at[pl.ds(_ea_off(gb + b), G)], ring.at[b],
                             gsem.at[b])

        def mloop(k2, _):
            for b in range(NBUF):
                j = k2 * NBUF + b
                pltpu.make_async_copy(ea.at[pl.ds(0, G)], ring.at[b],
                                      gsem.at[b]).wait()
                pltpu.async_copy(ring.at[b], acc.at[dstv.at[j]], ssem.at[b],
                                 add=True)
                bp = (b - 1) % NBUF
                @pl.when((j >= 1) & (j - 1 + NBUF < SCH))
                def _():
                    pltpu.make_async_copy(ea.at[pl.ds(0, G)], ring.at[bp],
                                          ssem.at[bp]).wait()
                    pltpu.async_copy(ea.at[pl.ds(_ea_off(gb + j - 1 + NBUF), G)],
                                     ring.at[bp], gsem.at[bp])
            return 0
        lax.fori_loop(0, SCH // NBUF, mloop, 0)

        for b in range(NBUF):
            pltpu.make_async_copy(ea.at[pl.ds(0, G)], ring.at[b],
                                  ssem.at[b]).wait()
        return 0
    lax.fori_loop(0, NSC_F, schunk, 0)
    plsc.subcore_barrier()

    @pl.when(c == 0)
    def _():
        pltpu.sync_copy(acc.at[pl.ds(zbase, RPT)], nfa.at[pl.ds(zbase, RPT)])
    @pl.when(c == 1)
    def _():
        pltpu.sync_copy(acc.at[pl.ds(zbase, RPT)], nfb.at[pl.ds(zbase, RPT)])


_bond_sc = functools.partial(
    pl.kernel,
    out_type=[jax.ShapeDtypeStruct((NPAD, BOND_FDIM), jnp.float32),
              jax.ShapeDtypeStruct((NPAD, BOND_FDIM), jnp.float32)],
    mesh=_MESH,
    compiler_params=pltpu.CompilerParams(use_tc_tiling_on_sc=False),
    scratch_types=[
        pltpu.VMEM_SHARED((NPAD, BOND_FDIM), jnp.float32),
        pltpu.VMEM((SCH, G), jnp.int32),
        pltpu.VMEM((NBUF, G, BOND_FDIM), jnp.float32),
        pltpu.SemaphoreType.DMA((NBUF,)),
        pltpu.SemaphoreType.DMA((NBUF,)),
    ],
)(_bond_body)


def kernel(x, edge_index, edge_attr, W_i, W_h, W_o):
    src = edge_index[0].astype(jnp.int32)
    dst = edge_index[1].astype(jnp.int32)

    src2 = jnp.pad(src, (0, EPAD - N_EDGES)).reshape(NGRP, G)
    dst2 = jnp.pad(dst, (0, EPAD - N_EDGES),
                   constant_values=DUMMY).reshape(NGRP, G)
    zq = jnp.zeros((RPT, QW), jnp.float32)
    zf = jnp.zeros((RPT, BOND_FDIM), jnp.float32)

    wi4 = jnp.pad(W_i, ((0, 0), (0, HPAD - HIDDEN)))
    wi4 = wi4.reshape(ATOM_FDIM, NQ, QW).transpose(1, 0, 2)
    wh1 = jnp.pad(W_h[:HIDDEN],
                  ((0, HPAD - HIDDEN), (0, HPAD - HIDDEN)))
    wh1 = wh1.reshape(NQ, QW, NQ, QW).transpose(2, 0, 1, 3)
    wh2 = jnp.pad(W_h[HIDDEN:], ((0, 0), (0, HPAD - HIDDEN)))
    wh2 = wh2.reshape(BOND_FDIM, NQ, QW).transpose(1, 0, 2)
    wo1 = W_o[:ATOM_FDIM]
    wo2 = jnp.pad(W_o[ATOM_FDIM:], ((0, HPAD - HIDDEN), (0, 0)))
    wo2 = wo2.reshape(NQ, QW, HIDDEN)

    inp, mq = _mm_init(x, wi4)
    nfa, nfb = _bond_sc(edge_attr, dst2, zf)

    for _ in range(DEPTH - 1):
        n4 = _segsum_sc(mq, src2, dst2, zq)
        mq = _mm_round(inp, nfa, nfb, n4, wh2, wh1)

    n4 = _segsum_sc(mq, src2, dst2, zq)
    return _mm_out(x, n4, wo1, wo2)


# speedup vs baseline: 1.3102x; 1.0903x over previous
import functools

import jax
import jax.numpy as jnp
from jax import lax
from jax.experimental import pallas as pl
from jax.experimental.pallas import tpu as pltpu
from jax.experimental.pallas import tpu_sc as plsc

N_NODES = 10000
N_EDGES = 320000
ATOM_FDIM = 128
BOND_FDIM = 16
HIDDEN = 300
HPAD = 320
NQ = 4
QW = HPAD // NQ
DEPTH = 3
BR = 1000
NB = N_NODES // BR

NC, NS = 2, 16
G = 64
EPAD = 327680
NGRP = EPAD // G
GPT = NGRP // NS
SCH = 32
NSC = GPT // SCH
NBUF = 4
NPAD = 10112
DUMMY = 10000
RPT = NPAD // NS
GPT_F = NGRP // (NC * NS)
NSC_F = GPT_F // SCH
NGRP_REAL = N_EDGES // G


def _mm_init_body(x_ref, wi_ref, inp_ref, mq_ref):
    v = jnp.dot(x_ref[...], wi_ref[0], preferred_element_type=jnp.float32)
    inp_ref[0] = v
    mq_ref[0] = jnp.maximum(v, 0.0)


def _mm_round_body(inp_ref, nfa_ref, nfb_ref, n4_ref, wh2_ref, wh1_ref, mq_ref):
    nf = nfa_ref[...] + nfb_ref[...]
    v = (inp_ref[0]
         + jnp.dot(nf, wh2_ref[0], preferred_element_type=jnp.float32))
    w = wh1_ref[0]
    for p in range(NQ):
        v = v + jnp.dot(n4_ref[p], w[p], preferred_element_type=jnp.float32)
    mq_ref[0] = jnp.maximum(v, 0.0)


def _mm_out_body(x_ref, n4_ref, wo1_ref, wo2_ref, o_ref):
    v = jnp.dot(x_ref[...], wo1_ref[...], preferred_element_type=jnp.float32)
    for p in range(NQ):
        v = v + jnp.dot(n4_ref[p], wo2_ref[p], preferred_element_type=jnp.float32)
    o_ref[...] = jnp.maximum(v, 0.0)


def _mm_init(x, wi4):
    return pl.pallas_call(
        _mm_init_body,
        grid=(NB, NQ),
        in_specs=[pl.BlockSpec((BR, ATOM_FDIM), lambda i, q: (i, 0)),
                  pl.BlockSpec((1, ATOM_FDIM, QW), lambda i, q: (q, 0, 0))],
        out_specs=[pl.BlockSpec((1, BR, QW), lambda i, q: (q, i, 0)),
                   pl.BlockSpec((1, BR, QW), lambda i, q: (q, i, 0))],
        out_shape=[jax.ShapeDtypeStruct((NQ, N_NODES, QW), jnp.float32),
                   jax.ShapeDtypeStruct((NQ, NPAD, QW), jnp.float32)],
    )(x, wi4)


def _mm_round(inp, nfa, nfb, n4, wh2, wh1):
    return pl.pallas_call(
        _mm_round_body,
        grid=(NB, NQ),
        in_specs=[pl.BlockSpec((1, BR, QW), lambda i, q: (q, i, 0)),
                  pl.BlockSpec((BR, BOND_FDIM), lambda i, q: (i, 0)),
                  pl.BlockSpec((BR, BOND_FDIM), lambda i, q: (i, 0)),
                  pl.BlockSpec((NQ, BR, QW), lambda i, q: (0, i, 0)),
                  pl.BlockSpec((1, BOND_FDIM, QW), lambda i, q: (q, 0, 0)),
                  pl.BlockSpec((1, NQ, QW, QW), lambda i, q: (q, 0, 0, 0))],
        out_specs=pl.BlockSpec((1, BR, QW), lambda i, q: (q, i, 0)),
        out_shape=jax.ShapeDtypeStruct((NQ, NPAD, QW), jnp.float32),
    )(inp, nfa, nfb, n4, wh2, wh1)


def _mm_out(x, n4, wo1, wo2):
    return pl.pallas_call(
        _mm_out_body,
        grid=(NB,),
        in_specs=[pl.BlockSpec((BR, ATOM_FDIM), lambda i: (i, 0)),
                  pl.BlockSpec((NQ, BR, QW), lambda i: (0, i, 0)),
                  pl.BlockSpec((ATOM_FDIM, HIDDEN), lambda i: (0, 0)),
                  pl.BlockSpec((NQ, QW, HIDDEN), lambda i: (0, 0, 0))],
        out_specs=pl.BlockSpec((BR, HIDDEN), lambda i: (i, 0)),
        out_shape=jax.ShapeDtypeStruct((N_NODES, HIDDEN), jnp.float32),
    )(x, n4, wo1, wo2)


_MESH = plsc.VectorSubcoreMesh(core_axis_name="c", subcore_axis_name="s")


def _segsum_body(mq, src2, dst2, zq, n4, tbl, acc, srcv, dstv, ring,
                 gsem, ssem, isem):
    c = lax.axis_index("c")
    s = lax.axis_index("s")
    zbase = s * RPT

    for q in range(NC):
        qq = c * NC + q

        pltpu.sync_copy(zq, acc.at[pl.ds(zbase, RPT)])
        pltpu.sync_copy(mq.at[qq, pl.ds(zbase, RPT)],
                        tbl.at[pl.ds(zbase, RPT)])
        plsc.subcore_barrier()

        gb0 = s * GPT

        def _idx_wait():
            pltpu.make_async_copy(src2.at[pl.ds(0, SCH)], srcv.at[0],
                                  isem).wait()
            pltpu.make_async_copy(dst2.at[pl.ds(0, SCH)], dstv.at[0],
                                  isem).wait()

        def _idx_fetch(m, ib):
            pltpu.async_copy(src2.at[pl.ds(gb0 + m * SCH, SCH)],
                             srcv.at[ib], isem)
            pltpu.async_copy(dst2.at[pl.ds(gb0 + m * SCH, SCH)],
                             dstv.at[ib], isem)

        _idx_fetch(0, 0)
        _idx_wait()
        _idx_fetch(1, 1)
        for b in range(NBUF):
            pltpu.async_copy(tbl.at[srcv.at[0, b]], ring.at[b], gsem.at[b])

        SPB = SCH // NBUF

        def mloop(k2, _):
            for b in range(NBUF):
                j = k2 * NBUF + b
                if b == 1:
                    @pl.when((k2 % SPB == SPB - 1) & (j + NBUF - 1 < GPT))
                    def _():
                        _idx_wait()
                if b == 0:
                    @pl.when((k2 % SPB == 0) & (k2 > 0) & (j + SCH < GPT))
                    def _():
                        m = j // SCH + 1
                        _idx_fetch(m, m % 2)
                pltpu.make_async_copy(mq.at[0, pl.ds(0, G)], ring.at[b],
                                      gsem.at[b]).wait()
                ibj = (j // SCH) % 2
                pltpu.async_copy(ring.at[b], acc.at[dstv.at[ibj, j % SCH]],
                                 ssem.at[b], add=True)
                bp = (b - 1) % NBUF
                @pl.when((j >= 1) & (j - 1 + NBUF < GPT))
                def _():
                    jn = j - 1 + NBUF
                    ibn = (jn // SCH) % 2
                    pltpu.make_async_copy(mq.at[0, pl.ds(0, G)],
                                          ring.at[bp], ssem.at[bp]).wait()
                    pltpu.async_copy(tbl.at[srcv.at[ibn, jn % SCH]],
                                     ring.at[bp], gsem.at[bp])
            return 0
        lax.fori_loop(0, GPT // NBUF, mloop, 0)

        for b in range(NBUF):
            pltpu.make_async_copy(mq.at[0, pl.ds(0, G)], ring.at[b],
                                  ssem.at[b]).wait()
        plsc.subcore_barrier()

        pltpu.sync_copy(acc.at[pl.ds(zbase, RPT)],
                        n4.at[qq, pl.ds(zbase, RPT)])
        plsc.subcore_barrier()


_segsum_sc = functools.partial(
    pl.kernel,
    out_type=jax.ShapeDtypeStruct((NQ, NPAD, QW), jnp.float32),
    mesh=_MESH,
    compiler_params=pltpu.CompilerParams(use_tc_tiling_on_sc=False),
    scratch_types=[
        pltpu.VMEM_SHARED((NPAD, QW), jnp.float32),
        pltpu.VMEM_SHARED((NPAD, QW), jnp.float32),
        pltpu.VMEM((2, SCH, G), jnp.int32),
        pltpu.VMEM((2, SCH, G), jnp.int32),
        pltpu.VMEM((NBUF, G, QW), jnp.float32),
        pltpu.SemaphoreType.DMA((NBUF,)),
        pltpu.SemaphoreType.DMA((NBUF,)),
        pltpu.SemaphoreType.DMA,
    ],
)(_segsum_body)


def _bond_body(ea, dst2, zf, nfa, nfb, acc, dstv, ring, gsem, ssem):
    c = lax.axis_index("c")
    s = lax.axis_index("s")
    zbase = s * RPT

    pltpu.sync_copy(zf, acc.at[pl.ds(zbase, RPT)])
    plsc.subcore_barrier()

    def _ea_off(g):
        return jnp.minimum(g, NGRP_REAL - 1) * G

    def schunk(sc_i, _):
        gb = (c * NS + s) * GPT_F + sc_i * SCH
        pltpu.sync_copy(dst2.at[pl.ds(gb, SCH)], dstv)

        for b in range(NBUF):
            pltpu.async_copy(ea.at[pl.ds(_ea_off(gb + b), G)], ring.at[b],
                             gsem.at[b])

        def mloop(k2, _):
            for b in range(NBUF):
                j = k2 * NBUF + b
                pltpu.make_async_copy(ea.at[pl.ds(0, G)], ring.at[b],
                                      gsem.at[b]).wait()
                pltpu.async_copy(ring.at[b], acc.at[dstv.at[j]], ssem.at[b],
                                 add=True)
                bp = (b - 1) % NBUF
                @pl.when((j >= 1) & (j - 1 + NBUF < SCH))
                def _():
                    pltpu.make_async_copy(ea.at[pl.ds(0, G)], ring.at[bp],
                                          ssem.at[bp]).wait()
                    pltpu.async_copy(ea.at[pl.ds(_ea_off(gb + j - 1 + NBUF), G)],
                                     ring.at[bp], gsem.at[bp])
            return 0
        lax.fori_loop(0, SCH // NBUF, mloop, 0)

        for b in range(NBUF):
            pltpu.make_async_copy(ea.at[pl.ds(0, G)], ring.at[b],
                                  ssem.at[b]).wait()
        return 0
    lax.fori_loop(0, NSC_F, schunk, 0)
    plsc.subcore_barrier()

    @pl.when(c == 0)
    def _():
        pltpu.sync_copy(acc.at[pl.ds(zbase, RPT)], nfa.at[pl.ds(zbase, RPT)])
    @pl.when(c == 1)
    def _():
        pltpu.sync_copy(acc.at[pl.ds(zbase, RPT)], nfb.at[pl.ds(zbase, RPT)])


_bond_sc = functools.partial(
    pl.kernel,
    out_type=[jax.ShapeDtypeStruct((NPAD, BOND_FDIM), jnp.float32),
              jax.ShapeDtypeStruct((NPAD, BOND_FDIM), jnp.float32)],
    mesh=_MESH,
    compiler_params=pltpu.CompilerParams(use_tc_tiling_on_sc=False),
    scratch_types=[
        pltpu.VMEM_SHARED((NPAD, BOND_FDIM), jnp.float32),
        pltpu.VMEM((SCH, G), jnp.int32),
        pltpu.VMEM((NBUF, G, BOND_FDIM), jnp.float32),
        pltpu.SemaphoreType.DMA((NBUF,)),
        pltpu.SemaphoreType.DMA((NBUF,)),
    ],
)(_bond_body)


def kernel(x, edge_index, edge_attr, W_i, W_h, W_o):
    src = edge_index[0].astype(jnp.int32)
    dst = edge_index[1].astype(jnp.int32)

    src2 = jnp.pad(src, (0, EPAD - N_EDGES)).reshape(NGRP, G)
    dst2 = jnp.pad(dst, (0, EPAD - N_EDGES),
                   constant_values=DUMMY).reshape(NGRP, G)
    zq = jnp.zeros((RPT, QW), jnp.float32)
    zf = jnp.zeros((RPT, BOND_FDIM), jnp.float32)

    wi4 = jnp.pad(W_i, ((0, 0), (0, HPAD - HIDDEN)))
    wi4 = wi4.reshape(ATOM_FDIM, NQ, QW).transpose(1, 0, 2)
    wh1 = jnp.pad(W_h[:HIDDEN],
                  ((0, HPAD - HIDDEN), (0, HPAD - HIDDEN)))
    wh1 = wh1.reshape(NQ, QW, NQ, QW).transpose(2, 0, 1, 3)
    wh2 = jnp.pad(W_h[HIDDEN:], ((0, 0), (0, HPAD - HIDDEN)))
    wh2 = wh2.reshape(BOND_FDIM, NQ, QW).transpose(1, 0, 2)
    wo1 = W_o[:ATOM_FDIM]
    wo2 = jnp.pad(W_o[ATOM_FDIM:], ((0, HPAD - HIDDEN), (0, 0)))
    wo2 = wo2.reshape(NQ, QW, HIDDEN)

    inp, mq = _mm_init(x, wi4)
    nfa, nfb = _bond_sc(edge_attr, dst2, zf)

    for _ in range(DEPTH - 1):
        n4 = _segsum_sc(mq, src2, dst2, zq)
        mq = _mm_round(inp, nfa, nfb, n4, wh2, wh1)

    n4 = _segsum_sc(mq, src2, dst2, zq)
    return _mm_out(x, n4, wo1, wo2)


# speedup vs baseline: 1.3187x; 1.0065x over previous
import functools

import jax
import jax.numpy as jnp
from jax import lax
from jax.experimental import pallas as pl
from jax.experimental.pallas import tpu as pltpu
from jax.experimental.pallas import tpu_sc as plsc

N_NODES = 10000
N_EDGES = 320000
ATOM_FDIM = 128
BOND_FDIM = 16
HIDDEN = 300
HPAD = 320
NQ = 4
QW = HPAD // NQ
DEPTH = 3
BR = 1000
NB = N_NODES // BR

NC, NS = 2, 16
G = 64
EPAD = 327680
NGRP = EPAD // G
GPT = NGRP // NS
SCH = 32
NSC = GPT // SCH
NBUF = 4
NPAD = 10112
DUMMY = 10000
RPT = NPAD // NS
GPT_F = NGRP // (NC * NS)
NSC_F = GPT_F // SCH
NGRP_REAL = N_EDGES // G


def _mm_init_body(x_ref, wi_ref, inp_ref, mq_ref):
    v = jnp.dot(x_ref[...], wi_ref[0], preferred_element_type=jnp.float32)
    inp_ref[0] = v
    mq_ref[0] = jnp.maximum(v, 0.0)


def _mm_round_body(inp_ref, nfa_ref, nfb_ref, n4_ref, wh2_ref, wh1_ref, mq_ref):
    nf = nfa_ref[...] + nfb_ref[...]
    v = (inp_ref[0]
         + jnp.dot(nf, wh2_ref[0], preferred_element_type=jnp.float32))
    w = wh1_ref[0]
    for p in range(NQ):
        v = v + jnp.dot(n4_ref[p], w[p], preferred_element_type=jnp.float32)
    mq_ref[0] = jnp.maximum(v, 0.0)


def _mm_out_body(x_ref, n4_ref, wo1_ref, wo2_ref, o_ref):
    v = jnp.dot(x_ref[...], wo1_ref[...], preferred_element_type=jnp.float32)
    for p in range(NQ):
        v = v + jnp.dot(n4_ref[p], wo2_ref[p], preferred_element_type=jnp.float32)
    o_ref[...] = jnp.maximum(v, 0.0)


def _mm_init(x, wi4):
    return pl.pallas_call(
        _mm_init_body,
        grid=(NB, NQ),
        in_specs=[pl.BlockSpec((BR, ATOM_FDIM), lambda i, q: (i, 0)),
                  pl.BlockSpec((1, ATOM_FDIM, QW), lambda i, q: (q, 0, 0))],
        out_specs=[pl.BlockSpec((1, BR, QW), lambda i, q: (q, i, 0)),
                   pl.BlockSpec((1, BR, QW), lambda i, q: (q, i, 0))],
        out_shape=[jax.ShapeDtypeStruct((NQ, N_NODES, QW), jnp.float32),
                   jax.ShapeDtypeStruct((NQ, NPAD, QW), jnp.float32)],
    )(x, wi4)


def _mm_round(inp, nfa, nfb, n4, wh2, wh1):
    return pl.pallas_call(
        _mm_round_body,
        grid=(NB, NQ),
        in_specs=[pl.BlockSpec((1, BR, QW), lambda i, q: (q, i, 0)),
                  pl.BlockSpec((BR, BOND_FDIM), lambda i, q: (i, 0)),
                  pl.BlockSpec((BR, BOND_FDIM), lambda i, q: (i, 0)),
                  pl.BlockSpec((NQ, BR, QW), lambda i, q: (0, i, 0)),
                  pl.BlockSpec((1, BOND_FDIM, QW), lambda i, q: (q, 0, 0)),
                  pl.BlockSpec((1, NQ, QW, QW), lambda i, q: (q, 0, 0, 0))],
        out_specs=pl.BlockSpec((1, BR, QW), lambda i, q: (q, i, 0)),
        out_shape=jax.ShapeDtypeStruct((NQ, NPAD, QW), jnp.float32),
    )(inp, nfa, nfb, n4, wh2, wh1)


def _mm_out(x, n4, wo1, wo2):
    return pl.pallas_call(
        _mm_out_body,
        grid=(NB,),
        in_specs=[pl.BlockSpec((BR, ATOM_FDIM), lambda i: (i, 0)),
                  pl.BlockSpec((NQ, BR, QW), lambda i: (0, i, 0)),
                  pl.BlockSpec((ATOM_FDIM, HIDDEN), lambda i: (0, 0)),
                  pl.BlockSpec((NQ, QW, HIDDEN), lambda i: (0, 0, 0))],
        out_specs=pl.BlockSpec((BR, HIDDEN), lambda i: (i, 0)),
        out_shape=jax.ShapeDtypeStruct((N_NODES, HIDDEN), jnp.float32),
    )(x, n4, wo1, wo2)


_MESH = plsc.VectorSubcoreMesh(core_axis_name="c", subcore_axis_name="s")


def _segsum_body(mq, src2, dst2, zq, n4, tbl, acc, srcv, dstv, ring,
                 gsem, ssem, isem):
    c = lax.axis_index("c")
    s = lax.axis_index("s")
    zbase = s * RPT

    for q in range(NC):
        qq = c * NC + q

        pltpu.sync_copy(zq, acc.at[pl.ds(zbase, RPT)])
        pltpu.sync_copy(mq.at[qq, pl.ds(zbase, RPT)],
                        tbl.at[pl.ds(zbase, RPT)])
        plsc.subcore_barrier()

        gb0 = s * GPT

        def _idx_wait():
            pltpu.make_async_copy(src2.at[pl.ds(0, SCH)], srcv.at[0],
                                  isem).wait()
            pltpu.make_async_copy(dst2.at[pl.ds(0, SCH)], dstv.at[0],
                                  isem).wait()

        def _idx_fetch(m, ib):
            pltpu.async_copy(src2.at[pl.ds(gb0 + m * SCH, SCH)],
                             srcv.at[ib], isem)
            pltpu.async_copy(dst2.at[pl.ds(gb0 + m * SCH, SCH)],
                             dstv.at[ib], isem)

        _idx_fetch(0, 0)
        _idx_wait()
        _idx_fetch(1, 1)
        for b in range(NBUF):
            pltpu.async_copy(tbl.at[srcv.at[0, b]], ring.at[b], gsem.at[b])

        SPB = SCH // NBUF

        def mloop(k2, _):
            for b in range(NBUF):
                j = k2 * NBUF + b
                if b == 1:
                    @pl.when((k2 % SPB == SPB - 1) & (j + NBUF - 1 < GPT))
                    def _():
                        _idx_wait()
                if b == 0:
                    @pl.when((k2 % SPB == 0) & (k2 > 0) & (j + SCH < GPT))
                    def _():
                        m = j // SCH + 1
                        _idx_fetch(m, m % 2)
                pltpu.make_async_copy(mq.at[0, pl.ds(0, G)], ring.at[b],
                                      gsem.at[b]).wait()
                ibj = (j // SCH) % 2
                pltpu.async_copy(ring.at[b], acc.at[dstv.at[ibj, j % SCH]],
                                 ssem.at[b], add=True)
                bp = (b - 1) % NBUF
                @pl.when((j >= 1) & (j - 1 + NBUF < GPT))
                def _():
                    jn = j - 1 + NBUF
                    ibn = (jn // SCH) % 2
                    pltpu.make_async_copy(mq.at[0, pl.ds(0, G)],
                                          ring.at[bp], ssem.at[bp]).wait()
                    pltpu.async_copy(tbl.at[srcv.at[ibn, jn % SCH]],
                                     ring.at[bp], gsem.at[bp])
            return 0
        lax.fori_loop(0, GPT // NBUF, mloop, 0)

        for b in range(NBUF):
            pltpu.make_async_copy(mq.at[0, pl.ds(0, G)], ring.at[b],
                                  ssem.at[b]).wait()
        plsc.subcore_barrier()

        pltpu.sync_copy(acc.at[pl.ds(zbase, RPT)],
                        n4.at[qq, pl.ds(zbase, RPT)])


_segsum_sc = functools.partial(
    pl.kernel,
    out_type=jax.ShapeDtypeStruct((NQ, NPAD, QW), jnp.float32),
    mesh=_MESH,
    compiler_params=pltpu.CompilerParams(use_tc_tiling_on_sc=False),
    scratch_types=[
        pltpu.VMEM_SHARED((NPAD, QW), jnp.float32),
        pltpu.VMEM_SHARED((NPAD, QW), jnp.float32),
        pltpu.VMEM((2, SCH, G), jnp.int32),
        pltpu.VMEM((2, SCH, G), jnp.int32),
        pltpu.VMEM((NBUF, G, QW), jnp.float32),
        pltpu.SemaphoreType.DMA((NBUF,)),
        pltpu.SemaphoreType.DMA((NBUF,)),
        pltpu.SemaphoreType.DMA,
    ],
)(_segsum_body)


def _bond_body(ea, dst2, zf, nfa, nfb, acc, dstv, ring, gsem, ssem):
    c = lax.axis_index("c")
    s = lax.axis_index("s")
    zbase = s * RPT

    pltpu.sync_copy(zf, acc.at[pl.ds(zbase, RPT)])
    plsc.subcore_barrier()

    def _ea_off(g):
        return jnp.minimum(g, NGRP_REAL - 1) * G

    def schunk(sc_i, _):
        gb = (c * NS + s) * GPT_F + sc_i * SCH
        pltpu.sync_copy(dst2.at[pl.ds(gb, SCH)], dstv)

        for b in range(NBUF):
            pltpu.async_copy(ea.at[pl.ds(_ea_off(gb + b), G)], ring.at[b],
                             gsem.at[b])

        def mloop(k2, _):
            for b in range(NBUF):
                j = k2 * NBUF + b
                pltpu.make_async_copy(ea.at[pl.ds(0, G)], ring.at[b],
                                      gsem.at[b]).wait()
                pltpu.async_copy(ring.at[b], acc.at[dstv.at[j]], ssem.at[b],
                                 add=True)
                bp = (b - 1) % NBUF
                @pl.when((j >= 1) & (j - 1 + NBUF < SCH))
                def _():
                    pltpu.make_async_copy(ea.at[pl.ds(0, G)], ring.at[bp],
                                          ssem.at[bp]).wait()
                    pltpu.async_copy(ea.at[pl.ds(_ea_off(gb + j - 1 + NBUF), G)],
                                     ring.at[bp], gsem.at[bp])
            return 0
        lax.fori_loop(0, SCH // NBUF, mloop, 0)

        for b in range(NBUF):
            pltpu.make_async_copy(ea.at[pl.ds(0, G)], ring.at[b],
                                  ssem.at[b]).wait()
        return 0
    lax.fori_loop(0, NSC_F, schunk, 0)
    plsc.subcore_barrier()

    @pl.when(c == 0)
    def _():
        pltpu.sync_copy(acc.at[pl.ds(zbase, RPT)], nfa.at[pl.ds(zbase, RPT)])
    @pl.when(c == 1)
    def _():
        pltpu.sync_copy(acc.at[pl.ds(zbase, RPT)], nfb.at[pl.ds(zbase, RPT)])


_bond_sc = functools.partial(
    pl.kernel,
    out_type=[jax.ShapeDtypeStruct((NPAD, BOND_FDIM), jnp.float32),
              jax.ShapeDtypeStruct((NPAD, BOND_FDIM), jnp.float32)],
    mesh=_MESH,
    compiler_params=pltpu.CompilerParams(use_tc_tiling_on_sc=False),
    scratch_types=[
        pltpu.VMEM_SHARED((NPAD, BOND_FDIM), jnp.float32),
        pltpu.VMEM((SCH, G), jnp.int32),
        pltpu.VMEM((NBUF, G, BOND_FDIM), jnp.float32),
        pltpu.SemaphoreType.DMA((NBUF,)),
        pltpu.SemaphoreType.DMA((NBUF,)),
    ],
)(_bond_body)


def kernel(x, edge_index, edge_attr, W_i, W_h, W_o):
    src = edge_index[0].astype(jnp.int32)
    dst = edge_index[1].astype(jnp.int32)

    src2 = jnp.pad(src, (0, EPAD - N_EDGES)).reshape(NGRP, G)
    dst2 = jnp.pad(dst, (0, EPAD - N_EDGES),
                   constant_values=DUMMY).reshape(NGRP, G)
    zq = jnp.zeros((RPT, QW), jnp.float32)
    zf = jnp.zeros((RPT, BOND_FDIM), jnp.float32)

    wi4 = jnp.pad(W_i, ((0, 0), (0, HPAD - HIDDEN)))
    wi4 = wi4.reshape(ATOM_FDIM, NQ, QW).transpose(1, 0, 2)
    wh1 = jnp.pad(W_h[:HIDDEN],
                  ((0, HPAD - HIDDEN), (0, HPAD - HIDDEN)))
    wh1 = wh1.reshape(NQ, QW, NQ, QW).transpose(2, 0, 1, 3)
    wh2 = jnp.pad(W_h[HIDDEN:], ((0, 0), (0, HPAD - HIDDEN)))
    wh2 = wh2.reshape(BOND_FDIM, NQ, QW).transpose(1, 0, 2)
    wo1 = W_o[:ATOM_FDIM]
    wo2 = jnp.pad(W_o[ATOM_FDIM:], ((0, HPAD - HIDDEN), (0, 0)))
    wo2 = wo2.reshape(NQ, QW, HIDDEN)

    inp, mq = _mm_init(x, wi4)
    nfa, nfb = _bond_sc(edge_attr, dst2, zf)

    for _ in range(DEPTH - 1):
        n4 = _segsum_sc(mq, src2, dst2, zq)
        mq = _mm_round(inp, nfa, nfb, n4, wh2, wh1)

    n4 = _segsum_sc(mq, src2, dst2, zq)
    return _mm_out(x, n4, wo1, wo2)
